# baseline (speedup 1.0000x reference)
import functools

import jax
import jax.numpy as jnp
from jax.experimental import pallas as pl
from jax.experimental.pallas import tpu as pltpu

_VMEM_LIMIT = 48 * 1024 * 1024
_EPS = 1e-5
_SLOT = 12


def _stats_kernel(x_ref, sum_ref, sq_ref):
    @pl.when(pl.program_id(0) == 0)
    def _init():
        sum_ref[...] = jnp.zeros_like(sum_ref)
        sq_ref[...] = jnp.zeros_like(sq_ref)

    xv = x_ref[...].astype(jnp.float32)
    sum_ref[...] += jnp.sum(xv, axis=0)
    sq_ref[...] += jnp.sum(xv * xv, axis=0)


def _channel_stats(x_ncdhw):
    N, C = x_ncdhw.shape[:2]
    S = x_ncdhw.size // (N * C)
    x3 = x_ncdhw.reshape(N, C, S)
    bn = 4
    while N % bn:
        bn //= 2

    sums, sq = pl.pallas_call(
        _stats_kernel,
        out_shape=(jax.ShapeDtypeStruct((C, S), jnp.float32),
                   jax.ShapeDtypeStruct((C, S), jnp.float32)),
        grid=(N // bn,),
        in_specs=[pl.BlockSpec((bn, C, S), lambda i: (i, 0, 0))],
        out_specs=(pl.BlockSpec((C, S), lambda i: (0, 0)),
                   pl.BlockSpec((C, S), lambda i: (0, 0))),
        compiler_params=pltpu.CompilerParams(
            dimension_semantics=("arbitrary",),
            vmem_limit_bytes=_VMEM_LIMIT),
    )(x3)

    return sums.sum(axis=1), sq.sum(axis=1)


def _bn_coeffs(ch_sum, ch_sq, count, gamma, beta):
    mean = ch_sum / count
    var = ch_sq / count - mean * mean
    scale = gamma.astype(jnp.float32) * jax.lax.rsqrt(var + _EPS)
    shift = beta.astype(jnp.float32) - mean * scale
    return scale, shift


def _band_starts(W, Wt):
    starts = []
    for t in range(W // Wt):
        b = max(0, min(t * Wt - 2, W - _SLOT))
        starts.append(b - b % 4)
    return starts


def _wexpand_kernel(w_ref, out_ref, *, W, Wt, C):
    T = W // Wt
    starts = _band_starts(W, Wt)
    out_ref[...] = jnp.zeros_like(out_ref)
    for t in range(T):
        for kd in range(3):
            for kh in range(3):
                for kw in range(3):
                    for wl in range(Wt):
                        wg = t * Wt + wl + kw - 1
                        if wg < 0 or wg >= W:
                            continue
                        r = (kh * _SLOT + (wg - starts[t])) * C
                        out_ref[t, kd, r:r + C, wl * C:(wl + 1) * C] = \
                            w_ref[kd, kh, kw]


def _expand_weight(w_dhwio, W, Wt):
    KD, KH, KW, Ci, Co = w_dhwio.shape
    T = W // Wt
    return pl.pallas_call(
        functools.partial(_wexpand_kernel, W=W, Wt=Wt, C=Ci),
        out_shape=jax.ShapeDtypeStruct((T, KD, KH * _SLOT * Ci, Wt * Co),
                                       w_dhwio.dtype),
        compiler_params=pltpu.CompilerParams(
            vmem_limit_bytes=_VMEM_LIMIT),
    )(w_dhwio)


def _fused_conv_kernel(x_ref, scale_ref, shift_ref, w_ref, b_ref, *rest,
                       D, H, W, C, Wt, compute_stats, add_residual):
    i = 0
    res_ref = None
    if add_residual:
        res_ref = rest[i]; i += 1
    out_ref = rest[i]; i += 1
    sum_ref = sq_ref = None
    if compute_stats:
        sum_ref, sq_ref = rest[i], rest[i + 1]; i += 2
    lhs_ref, acc_ref = rest[i], rest[i + 1]

    T = W // Wt
    BK = _SLOT * C
    M = D * H
    starts = _band_starts(W, Wt)
    zrow = jnp.zeros((1, BK), jnp.float32)

    if compute_stats:
        @pl.when(pl.program_id(0) == 0)
        def _init():
            sum_ref[...] = jnp.zeros_like(sum_ref)
            sq_ref[...] = jnp.zeros_like(sq_ref)

    for t in range(T):
        NS = slice(t * Wt * C, (t + 1) * Wt * C)
        WIN = slice(starts[t] * C, starts[t] * C + BK)

        for d in range(D):
            xr = x_ref[0, d, :, WIN]
            xs = jnp.maximum(
                xr.astype(jnp.float32) * scale_ref[0, WIN] + shift_ref[0, WIN],
                0.0)
            r = d * H
            lhs_ref[t, r:r + 1, 0:BK] = zrow
            lhs_ref[t, r + 1:r + H, 0:BK] = xs[0:H - 1]
            lhs_ref[t, r:r + H, BK:2 * BK] = xs
            lhs_ref[t, r:r + H - 1, 2 * BK:3 * BK] = xs[1:H]
            lhs_ref[t, r + H - 1:r + H, 2 * BK:3 * BK] = zrow

        acc_ref[:, NS] = jnp.dot(lhs_ref[t, :, :], w_ref[t, 1],
                                 preferred_element_type=jnp.float32)
        acc_ref[H:M, NS] += jnp.dot(lhs_ref[t, 0:M - H, :], w_ref[t, 0],
                                    preferred_element_type=jnp.float32)
        acc_ref[0:M - H, NS] += jnp.dot(lhs_ref[t, H:M, :], w_ref[t, 2],
                                        preferred_element_type=jnp.float32)

        if compute_stats:
            s16 = jnp.zeros((H, Wt * C), jnp.float32)
            q16 = jnp.zeros((H, Wt * C), jnp.float32)
        for d in range(D):
            y = acc_ref[d * H:(d + 1) * H, NS] + b_ref[0, NS]
            if compute_stats:
                s16 = s16 + y
                q16 = q16 + y * y
            if add_residual:
                y = y + res_ref[0, d, :, NS].astype(jnp.float32)
            out_ref[0, d, :, NS] = y.astype(out_ref.dtype)
        if compute_stats:
            sum_ref[:, NS] += jnp.sum(s16, axis=0, keepdims=True)
            sq_ref[:, NS] += jnp.sum(q16, axis=0, keepdims=True)


def _fused_conv(xin, scale, shift, w_dhwio, bias, *, C, out_dtype,
                residual=None, compute_stats=False):
    N, D, H, WC = xin.shape
    W = WC // C
    Wt = 8 if W % 8 == 0 else W
    T = W // Wt

    wexp = _expand_weight(w_dhwio, W, Wt).astype(jnp.float32)
    scale_t = jnp.tile(scale.astype(jnp.float32), W).reshape(1, WC)
    shift_t = jnp.tile(shift.astype(jnp.float32), W).reshape(1, WC)
    bias_t = jnp.tile(bias.astype(jnp.float32), W).reshape(1, WC)

    body = functools.partial(
        _fused_conv_kernel, D=D, H=H, W=W, C=C, Wt=Wt,
        compute_stats=compute_stats, add_residual=residual is not None)

    in_specs = [
        pl.BlockSpec((1, D, H, WC), lambda n: (n, 0, 0, 0)),
        pl.BlockSpec((1, WC), lambda n: (0, 0)),
        pl.BlockSpec((1, WC), lambda n: (0, 0)),
        pl.BlockSpec(wexp.shape, lambda n: (0, 0, 0, 0)),
        pl.BlockSpec((1, WC), lambda n: (0, 0)),
    ]
    args = [xin, scale_t, shift_t, wexp, bias_t]
    if residual is not None:
        in_specs.append(pl.BlockSpec((1, D, H, WC), lambda n: (n, 0, 0, 0)))
        args.append(residual)

    y_shape = jax.ShapeDtypeStruct((N, D, H, WC), out_dtype)
    y_spec = pl.BlockSpec((1, D, H, WC), lambda n: (n, 0, 0, 0))
    if compute_stats:
        stat_shape = jax.ShapeDtypeStruct((1, WC), jnp.float32)
        stat_spec = pl.BlockSpec((1, WC), lambda n: (0, 0))
        out_shape = (y_shape, stat_shape, stat_shape)
        out_specs = (y_spec, stat_spec, stat_spec)
    else:
        out_shape = y_shape
        out_specs = y_spec

    return pl.pallas_call(
        body,
        out_shape=out_shape,
        grid=(N,),
        in_specs=in_specs,
        out_specs=out_specs,
        scratch_shapes=[
            pltpu.VMEM((T, D * H, 3 * _SLOT * C), jnp.float32),
            pltpu.VMEM((D * H, WC), jnp.float32),
        ],
        compiler_params=pltpu.CompilerParams(
            dimension_semantics=("arbitrary",),
            vmem_limit_bytes=_VMEM_LIMIT),
    )(*args)


def kernel(x, gamma1, beta1, w1, b1, gamma2, beta2, w2, b2):
    N, C, D, H, W = x.shape
    count = N * D * H * W
    x_f = jnp.transpose(x, (0, 2, 3, 4, 1)).reshape(N, D, H, W * C)

    s1, q1 = _channel_stats(x)
    scale1, shift1 = _bn_coeffs(s1, q1, count, gamma1, beta1)

    h1, hsum, hsq = _fused_conv(
        x_f, scale1, shift1, w1, b1, C=C, out_dtype=jnp.bfloat16,
        compute_stats=True)

    s2 = hsum.reshape(W, C).sum(axis=0)
    q2 = hsq.reshape(W, C).sum(axis=0)
    scale2, shift2 = _bn_coeffs(s2, q2, count, gamma2, beta2)

    out = _fused_conv(
        h1, scale2, shift2, w2, b2, C=C, out_dtype=x.dtype,
        residual=x_f, compute_stats=False)

    return out.reshape(N, D, H, W, C).transpose(0, 4, 1, 2, 3)


# speedup vs baseline: 1.1280x; 1.1280x over previous
import functools

import jax
import jax.numpy as jnp
from jax.experimental import pallas as pl
from jax.experimental.pallas import tpu as pltpu

_VMEM_LIMIT = 48 * 1024 * 1024
_EPS = 1e-5
_SLOT = 12


def _stats_kernel(x_ref, sum_ref, sq_ref):
    @pl.when(pl.program_id(0) == 0)
    def _init():
        sum_ref[...] = jnp.zeros_like(sum_ref)
        sq_ref[...] = jnp.zeros_like(sq_ref)

    L = x_ref.shape[-1]
    xv = x_ref[...].astype(jnp.float32).reshape(-1, L)
    sum_ref[...] += jnp.sum(xv, axis=0, keepdims=True)
    sq_ref[...] += jnp.sum(xv * xv, axis=0, keepdims=True)


def _channel_stats(x_f, C):
    N, D, H, L = x_f.shape
    bn = 8
    while N % bn:
        bn //= 2

    sums, sq = pl.pallas_call(
        _stats_kernel,
        out_shape=(jax.ShapeDtypeStruct((1, L), jnp.float32),
                   jax.ShapeDtypeStruct((1, L), jnp.float32)),
        grid=(N // bn,),
        in_specs=[pl.BlockSpec((bn, D, H, L), lambda i: (i, 0, 0, 0))],
        out_specs=(pl.BlockSpec((1, L), lambda i: (0, 0)),
                   pl.BlockSpec((1, L), lambda i: (0, 0))),
        compiler_params=pltpu.CompilerParams(
            dimension_semantics=("arbitrary",),
            vmem_limit_bytes=_VMEM_LIMIT),
    )(x_f)

    ch_sum = sums.reshape(L // C, C).sum(axis=0)
    ch_sq = sq.reshape(L // C, C).sum(axis=0)
    return ch_sum, ch_sq


def _bn_coeffs(ch_sum, ch_sq, count, gamma, beta):
    mean = ch_sum / count
    var = ch_sq / count - mean * mean
    scale = gamma.astype(jnp.float32) * jax.lax.rsqrt(var + _EPS)
    shift = beta.astype(jnp.float32) - mean * scale
    return scale, shift


def _band_starts(W, Wt):
    starts = []
    for t in range(W // Wt):
        b = max(0, min(t * Wt - 2, W - _SLOT))
        starts.append(b - b % 4)
    return starts


def _wexpand_kernel(w_ref, out_ref, *, W, Wt, C):
    T = W // Wt
    starts = _band_starts(W, Wt)
    out_ref[...] = jnp.zeros_like(out_ref)
    for t in range(T):
        for kd in range(3):
            for kh in range(3):
                for kw in range(3):
                    for wl in range(Wt):
                        wg = t * Wt + wl + kw - 1
                        if wg < 0 or wg >= W:
                            continue
                        r = (kh * _SLOT + (wg - starts[t])) * C
                        out_ref[t, kd, r:r + C, wl * C:(wl + 1) * C] = \
                            w_ref[kd, kh, kw]


def _expand_weight(w_dhwio, W, Wt):
    KD, KH, KW, Ci, Co = w_dhwio.shape
    T = W // Wt
    return pl.pallas_call(
        functools.partial(_wexpand_kernel, W=W, Wt=Wt, C=Ci),
        out_shape=jax.ShapeDtypeStruct((T, KD, KH * _SLOT * Ci, Wt * Co),
                                       w_dhwio.dtype),
        compiler_params=pltpu.CompilerParams(
            vmem_limit_bytes=_VMEM_LIMIT),
    )(w_dhwio)


def _fused_conv_kernel(x_ref, scale_ref, shift_ref, w_ref, b_ref, *rest,
                       D, H, W, C, Wt, compute_stats, add_residual):
    i = 0
    res_ref = None
    if add_residual:
        res_ref = rest[i]; i += 1
    out_ref = rest[i]; i += 1
    sum_ref = sq_ref = None
    if compute_stats:
        sum_ref, sq_ref = rest[i], rest[i + 1]; i += 2
    lhs_refs = rest[i:i + 2]
    acc_refs = rest[i + 2:i + 4]

    T = W // Wt
    BK = _SLOT * C
    M = D * H
    starts = _band_starts(W, Wt)
    zrow = jnp.zeros((1, BK), jnp.float32)

    if compute_stats:
        @pl.when(pl.program_id(0) == 0)
        def _init():
            sum_ref[...] = jnp.zeros_like(sum_ref)
            sq_ref[...] = jnp.zeros_like(sq_ref)

    for t in range(T):
        lhs_ref = lhs_refs[t]
        acc_ref = acc_refs[t]
        NS = slice(t * Wt * C, (t + 1) * Wt * C)
        WIN = slice(starts[t] * C, starts[t] * C + BK)

        for d in range(D):
            xr = x_ref[0, d, :, WIN]
            xs = jnp.maximum(
                xr.astype(jnp.float32) * scale_ref[0, WIN] + shift_ref[0, WIN],
                0.0)
            r = d * H
            lhs_ref[r:r + 1, 0:BK] = zrow
            lhs_ref[r + 1:r + H, 0:BK] = xs[0:H - 1]
            lhs_ref[r:r + H, BK:2 * BK] = xs
            lhs_ref[r:r + H - 1, 2 * BK:3 * BK] = xs[1:H]
            lhs_ref[r + H - 1:r + H, 2 * BK:3 * BK] = zrow

        acc_ref[...] = jnp.dot(lhs_ref[:, :], w_ref[t, 1],
                                 preferred_element_type=jnp.float32)
        acc_ref[H:M, :] += jnp.dot(lhs_ref[0:M - H, :], w_ref[t, 0],
                                    preferred_element_type=jnp.float32)
        acc_ref[0:M - H, :] += jnp.dot(lhs_ref[H:M, :], w_ref[t, 2],
                                        preferred_element_type=jnp.float32)

        if compute_stats:
            s16 = jnp.zeros((H, Wt * C), jnp.float32)
            q16 = jnp.zeros((H, Wt * C), jnp.float32)
        for d in range(D):
            y = acc_ref[d * H:(d + 1) * H, :] + b_ref[0, NS]
            if compute_stats:
                s16 = s16 + y
                q16 = q16 + y * y
            if add_residual:
                y = y + res_ref[0, d, :, NS].astype(jnp.float32)
            out_ref[0, d, :, NS] = y.astype(out_ref.dtype)
        if compute_stats:
            sum_ref[:, NS] += jnp.sum(s16, axis=0, keepdims=True)
            sq_ref[:, NS] += jnp.sum(q16, axis=0, keepdims=True)


def _fused_conv(xin, scale, shift, w_dhwio, bias, *, C, out_dtype,
                residual=None, compute_stats=False):
    N, D, H, WC = xin.shape
    W = WC // C
    Wt = 8 if W % 8 == 0 else W
    T = W // Wt

    wexp = _expand_weight(w_dhwio, W, Wt).astype(jnp.float32)
    scale_t = jnp.tile(scale.astype(jnp.float32), W).reshape(1, WC)
    shift_t = jnp.tile(shift.astype(jnp.float32), W).reshape(1, WC)
    bias_t = jnp.tile(bias.astype(jnp.float32), W).reshape(1, WC)

    body = functools.partial(
        _fused_conv_kernel, D=D, H=H, W=W, C=C, Wt=Wt,
        compute_stats=compute_stats, add_residual=residual is not None)

    in_specs = [
        pl.BlockSpec((1, D, H, WC), lambda n: (n, 0, 0, 0)),
        pl.BlockSpec((1, WC), lambda n: (0, 0)),
        pl.BlockSpec((1, WC), lambda n: (0, 0)),
        pl.BlockSpec(wexp.shape, lambda n: (0, 0, 0, 0)),
        pl.BlockSpec((1, WC), lambda n: (0, 0)),
    ]
    args = [xin, scale_t, shift_t, wexp, bias_t]
    if residual is not None:
        in_specs.append(pl.BlockSpec((1, D, H, WC), lambda n: (n, 0, 0, 0)))
        args.append(residual)

    y_shape = jax.ShapeDtypeStruct((N, D, H, WC), out_dtype)
    y_spec = pl.BlockSpec((1, D, H, WC), lambda n: (n, 0, 0, 0))
    if compute_stats:
        stat_shape = jax.ShapeDtypeStruct((1, WC), jnp.float32)
        stat_spec = pl.BlockSpec((1, WC), lambda n: (0, 0))
        out_shape = (y_shape, stat_shape, stat_shape)
        out_specs = (y_spec, stat_spec, stat_spec)
    else:
        out_shape = y_shape
        out_specs = y_spec

    return pl.pallas_call(
        body,
        out_shape=out_shape,
        grid=(N,),
        in_specs=in_specs,
        out_specs=out_specs,
        scratch_shapes=[
            pltpu.VMEM((D * H, 3 * _SLOT * C), jnp.float32),
            pltpu.VMEM((D * H, 3 * _SLOT * C), jnp.float32),
            pltpu.VMEM((D * H, Wt * C), jnp.float32),
            pltpu.VMEM((D * H, Wt * C), jnp.float32),
        ],
        compiler_params=pltpu.CompilerParams(
            dimension_semantics=("arbitrary",),
            vmem_limit_bytes=_VMEM_LIMIT),
    )(*args)


def kernel(x, gamma1, beta1, w1, b1, gamma2, beta2, w2, b2):
    N, C, D, H, W = x.shape
    count = N * D * H * W
    x_f = jnp.transpose(x, (0, 2, 3, 4, 1)).reshape(N, D, H, W * C)

    s1, q1 = _channel_stats(x_f, C)
    scale1, shift1 = _bn_coeffs(s1, q1, count, gamma1, beta1)

    h1, hsum, hsq = _fused_conv(
        x_f, scale1, shift1, w1, b1, C=C, out_dtype=jnp.bfloat16,
        compute_stats=True)

    s2 = hsum.reshape(W, C).sum(axis=0)
    q2 = hsq.reshape(W, C).sum(axis=0)
    scale2, shift2 = _bn_coeffs(s2, q2, count, gamma2, beta2)

    out = _fused_conv(
        h1, scale2, shift2, w2, b2, C=C, out_dtype=x.dtype,
        residual=x_f, compute_stats=False)

    return out.reshape(N, D, H, W, C).transpose(0, 4, 1, 2, 3)


# speedup vs baseline: 1.1541x; 1.0231x over previous
import functools

import jax
import jax.numpy as jnp
from jax.experimental import pallas as pl
from jax.experimental.pallas import tpu as pltpu

_VMEM_LIMIT = 48 * 1024 * 1024
_EPS = 1e-5
_SLOT = 12


def _stats_kernel(x_ref, sum_ref, sq_ref):
    @pl.when(pl.program_id(0) == 0)
    def _init():
        sum_ref[...] = jnp.zeros_like(sum_ref)
        sq_ref[...] = jnp.zeros_like(sq_ref)

    L = x_ref.shape[-1]
    xv = x_ref[...].astype(jnp.float32).reshape(-1, L)
    sum_ref[...] += jnp.sum(xv, axis=0, keepdims=True)
    sq_ref[...] += jnp.sum(xv * xv, axis=0, keepdims=True)


def _channel_stats(x_f, C):
    N, D, H, L = x_f.shape
    bn = 8
    while N % bn:
        bn //= 2

    sums, sq = pl.pallas_call(
        _stats_kernel,
        out_shape=(jax.ShapeDtypeStruct((1, L), jnp.float32),
                   jax.ShapeDtypeStruct((1, L), jnp.float32)),
        grid=(N // bn,),
        in_specs=[pl.BlockSpec((bn, D, H, L), lambda i: (i, 0, 0, 0))],
        out_specs=(pl.BlockSpec((1, L), lambda i: (0, 0)),
                   pl.BlockSpec((1, L), lambda i: (0, 0))),
        compiler_params=pltpu.CompilerParams(
            dimension_semantics=("arbitrary",),
            vmem_limit_bytes=_VMEM_LIMIT),
    )(x_f)

    ch_sum = sums.reshape(L // C, C).sum(axis=0)
    ch_sq = sq.reshape(L // C, C).sum(axis=0)
    return ch_sum, ch_sq


def _bn_coeffs(ch_sum, ch_sq, count, gamma, beta):
    mean = ch_sum / count
    var = ch_sq / count - mean * mean
    scale = gamma.astype(jnp.float32) * jax.lax.rsqrt(var + _EPS)
    shift = beta.astype(jnp.float32) - mean * scale
    return scale, shift


def _band_starts(W, Wt):
    starts = []
    for t in range(W // Wt):
        b = max(0, min(t * Wt - 2, W - _SLOT))
        starts.append(b - b % 4)
    return starts


def _wexpand_kernel(w_ref, out_ref, *, W, Wt, C):
    T = W // Wt
    starts = _band_starts(W, Wt)
    out_ref[...] = jnp.zeros_like(out_ref)
    for t in range(T):
        for kd in range(3):
            for kh in range(3):
                for kw in range(3):
                    for wl in range(Wt):
                        wg = t * Wt + wl + kw - 1
                        if wg < 0 or wg >= W:
                            continue
                        r = (kh * _SLOT + (wg - starts[t])) * C
                        out_ref[t, kd, r:r + C, wl * C:(wl + 1) * C] = \
                            w_ref[kd, kh, kw]


def _expand_weight(w_dhwio, W, Wt):
    KD, KH, KW, Ci, Co = w_dhwio.shape
    T = W // Wt
    return pl.pallas_call(
        functools.partial(_wexpand_kernel, W=W, Wt=Wt, C=Ci),
        out_shape=jax.ShapeDtypeStruct((T, KD, KH * _SLOT * Ci, Wt * Co),
                                       w_dhwio.dtype),
        compiler_params=pltpu.CompilerParams(
            vmem_limit_bytes=_VMEM_LIMIT),
    )(w_dhwio)


def _fused_conv_kernel(x_ref, scale_ref, shift_ref, w_ref, b_ref, *rest,
                       D, H, W, C, Wt, compute_stats, add_residual):
    i = 0
    res_ref = None
    if add_residual:
        res_ref = rest[i]; i += 1
    out_ref = rest[i]; i += 1
    sum_ref = sq_ref = None
    if compute_stats:
        sum_ref, sq_ref = rest[i], rest[i + 1]; i += 2
    lhs_refs = rest[i:i + 2]
    acc_refs = rest[i + 2:i + 4]

    T = W // Wt
    BK = _SLOT * C
    M = D * H
    starts = _band_starts(W, Wt)
    zrow = jnp.zeros((1, BK), jnp.float32)

    if compute_stats:
        @pl.when(pl.program_id(0) == 0)
        def _init():
            sum_ref[...] = jnp.zeros_like(sum_ref)
            sq_ref[...] = jnp.zeros_like(sq_ref)

    for t in range(T):
        lhs_ref = lhs_refs[t]
        acc_ref = acc_refs[t]
        NS = slice(t * Wt * C, (t + 1) * Wt * C)
        WIN = slice(starts[t] * C, starts[t] * C + BK)

        for d in range(D):
            xr = x_ref[0, d, :, WIN]
            xs = jnp.maximum(
                xr.astype(jnp.float32) * scale_ref[0, WIN] + shift_ref[0, WIN],
                0.0)
            r = d * H
            lhs_ref[r:r + 1, 0:BK] = zrow
            lhs_ref[r + 1:r + H, 0:BK] = xs[0:H - 1]
            lhs_ref[r:r + H, BK:2 * BK] = xs
            lhs_ref[r:r + H - 1, 2 * BK:3 * BK] = xs[1:H]
            lhs_ref[r + H - 1:r + H, 2 * BK:3 * BK] = zrow

        acc_ref[...] = jnp.dot(lhs_ref[:, :], w_ref[t, 1],
                                 preferred_element_type=jnp.float32)
        acc_ref[H:M, :] += jnp.dot(lhs_ref[0:M - H, :], w_ref[t, 0],
                                    preferred_element_type=jnp.float32)
        acc_ref[0:M - H, :] += jnp.dot(lhs_ref[H:M, :], w_ref[t, 2],
                                        preferred_element_type=jnp.float32)

        if compute_stats:
            s16 = jnp.zeros((H, Wt * C), jnp.float32)
            q16 = jnp.zeros((H, Wt * C), jnp.float32)
        for d in range(D):
            y = acc_ref[d * H:(d + 1) * H, :] + b_ref[0, NS]
            if compute_stats:
                s16 = s16 + y
                q16 = q16 + y * y
            if add_residual:
                y = y + res_ref[0, d, :, NS].astype(jnp.float32)
            out_ref[0, d, :, NS] = y.astype(out_ref.dtype)
        if compute_stats:
            sum_ref[:, NS] += jnp.sum(s16, axis=0, keepdims=True)
            sq_ref[:, NS] += jnp.sum(q16, axis=0, keepdims=True)


def _fused_conv(xin, scale, shift, w_dhwio, bias, *, C, out_dtype,
                residual=None, compute_stats=False):
    N, D, H, WC = xin.shape
    W = WC // C
    Wt = 8 if W % 8 == 0 else W
    T = W // Wt

    wexp = _expand_weight(w_dhwio, W, Wt).astype(jnp.float32)
    scale_t = jnp.tile(scale.astype(jnp.float32), W).reshape(1, WC)
    shift_t = jnp.tile(shift.astype(jnp.float32), W).reshape(1, WC)
    bias_t = jnp.tile(bias.astype(jnp.float32), W).reshape(1, WC)

    body = functools.partial(
        _fused_conv_kernel, D=D, H=H, W=W, C=C, Wt=Wt,
        compute_stats=compute_stats, add_residual=residual is not None)

    in_specs = [
        pl.BlockSpec((1, D, H, WC), lambda n: (n, 0, 0, 0)),
        pl.BlockSpec((1, WC), lambda n: (0, 0)),
        pl.BlockSpec((1, WC), lambda n: (0, 0)),
        pl.BlockSpec(wexp.shape, lambda n: (0, 0, 0, 0)),
        pl.BlockSpec((1, WC), lambda n: (0, 0)),
    ]
    args = [xin, scale_t, shift_t, wexp, bias_t]
    if residual is not None:
        in_specs.append(pl.BlockSpec((1, D, H, WC), lambda n: (n, 0, 0, 0)))
        args.append(residual)

    y_shape = jax.ShapeDtypeStruct((N, D, H, WC), out_dtype)
    y_spec = pl.BlockSpec((1, D, H, WC), lambda n: (n, 0, 0, 0))
    if compute_stats:
        stat_shape = jax.ShapeDtypeStruct((1, WC), jnp.float32)
        stat_spec = pl.BlockSpec((1, WC), lambda n: (0, 0))
        out_shape = (y_shape, stat_shape, stat_shape)
        out_specs = (y_spec, stat_spec, stat_spec)
    else:
        out_shape = y_shape
        out_specs = y_spec

    return pl.pallas_call(
        body,
        out_shape=out_shape,
        grid=(N,),
        in_specs=in_specs,
        out_specs=out_specs,
        scratch_shapes=[
            pltpu.VMEM((D * H, 3 * _SLOT * C), jnp.float32),
            pltpu.VMEM((D * H, 3 * _SLOT * C), jnp.float32),
            pltpu.VMEM((D * H, Wt * C), jnp.float32),
            pltpu.VMEM((D * H, Wt * C), jnp.float32),
        ],
        compiler_params=pltpu.CompilerParams(
            dimension_semantics=("arbitrary",),
            vmem_limit_bytes=_VMEM_LIMIT),
    )(*args)


def kernel(x, gamma1, beta1, w1, b1, gamma2, beta2, w2, b2):
    N, C, D, H, W = x.shape
    count = N * D * H * W
    x_f = jnp.transpose(x, (0, 2, 3, 4, 1)).reshape(N, D, H, W * C)

    s1, q1 = _channel_stats(x_f, C)
    scale1, shift1 = _bn_coeffs(s1, q1, count, gamma1, beta1)

    h1, hsum, hsq = _fused_conv(
        x_f, scale1, shift1, w1, b1, C=C, out_dtype=jnp.bfloat16,
        compute_stats=True)

    s2 = hsum.reshape(W, C).sum(axis=0)
    q2 = hsq.reshape(W, C).sum(axis=0)
    scale2, shift2 = _bn_coeffs(s2, q2, count, gamma2, beta2)

    out = _fused_conv(
        h1, scale2, shift2, w2, b2, C=C, out_dtype=jnp.bfloat16,
        residual=x_f, compute_stats=False)

    return out.reshape(N, D, H, W, C).transpose(
        0, 4, 1, 2, 3).astype(x.dtype)
